# baseline (speedup 1.0000x reference)
import functools

import jax
import jax.numpy as jnp
from jax import lax
from jax.experimental import pallas as pl
from jax.experimental.pallas import tpu as pltpu
from jax.experimental.pallas import tpu_sc as plsc

NUM_CORES = 2
NUM_SUBCORES = 16
NUM_WORKERS = NUM_CORES * NUM_SUBCORES
LANES = 16


@functools.lru_cache(maxsize=None)
def _build_gather_max(vocab, d_pad, batch, seq):
    half = seq // 2
    b_per_w = batch // NUM_WORKERS
    n_chunks = d_pad // LANES
    mesh = plsc.VectorSubcoreMesh(core_axis_name="c", subcore_axis_name="s")

    @functools.partial(
        pl.kernel,
        mesh=mesh,
        compiler_params=pltpu.CompilerParams(use_tc_tiling_on_sc=False),
        out_type=jax.ShapeDtypeStruct((batch, d_pad), jnp.float32),
        scratch_types=[
            pltpu.VMEM((2, 2, half), jnp.int32),
            pltpu.VMEM((2, 2, half, d_pad), jnp.float32),
            pltpu.VMEM((d_pad,), jnp.float32),
            pltpu.SemaphoreType.DMA,
            pltpu.SemaphoreType.DMA,
        ],
    )
    def gather_max(emb_hbm, idx_hbm, out_hbm, idx_v, rows_v, hrow_v, sem0, sem1):
        wid = lax.axis_index("s") * NUM_CORES + lax.axis_index("c")
        base = wid * b_per_w
        sems = (sem0, sem1)

        def issue(buf, i):
            pltpu.sync_copy(idx_hbm.at[i], idx_v.at[buf])
            pltpu.async_copy(emb_hbm.at[idx_v.at[buf, 0]], rows_v.at[buf, 0],
                             sems[buf])
            pltpu.async_copy(emb_hbm.at[idx_v.at[buf, 1]], rows_v.at[buf, 1],
                             sems[buf])

        def drain(buf):
            pltpu.make_async_copy(emb_hbm.at[idx_v.at[buf, 0]],
                                  rows_v.at[buf, 0], sems[buf]).wait()
            pltpu.make_async_copy(emb_hbm.at[idx_v.at[buf, 1]],
                                  rows_v.at[buf, 1], sems[buf]).wait()

        def compute_store(buf, i):
            r0 = rows_v.at[buf, 0]
            r1 = rows_v.at[buf, 1]

            def body(r, carry):
                out = []
                for c in range(n_chunks):
                    sl = pl.ds(c * LANES, LANES)
                    m = jnp.maximum(r0[r, sl], r1[r, sl])
                    out.append(jnp.maximum(carry[c], m))
                return tuple(out)

            neg_inf = jnp.full((LANES,), -jnp.inf, jnp.float32)
            acc = lax.fori_loop(0, half, body, (neg_inf,) * n_chunks)
            for c in range(n_chunks):
                hrow_v[pl.ds(c * LANES, LANES)] = acc[c]
            pltpu.sync_copy(hrow_v, out_hbm.at[i])

        issue(0, base)

        def loop_body(g, carry):
            i0 = base + 2 * g
            issue(1, i0 + 1)
            drain(0)
            compute_store(0, i0)

            @pl.when(g < b_per_w // 2 - 1)
            def _():
                issue(0, i0 + 2)

            drain(1)
            compute_store(1, i0 + 1)
            return carry

        lax.fori_loop(0, b_per_w // 2, loop_body, 0)

    return gather_max


def _tp_body(r0, e_ref, o_ref):
    t = jnp.transpose(e_ref[r0:, :], (1, 0))
    n, d = t.shape
    pad = o_ref.shape[1] - d
    if pad:
        t = jnp.concatenate([t, jnp.zeros((n, pad), jnp.float32)], axis=1)
    o_ref[...] = t


def _pad_chunk(emb_t, r0, h_blk, d_out):
    vocab = emb_t.shape[1]
    rows = 2048
    return pl.pallas_call(
        functools.partial(_tp_body, r0),
        grid=(-(-vocab // rows),),
        in_specs=[pl.BlockSpec((h_blk, rows), lambda i: (0, i))],
        out_specs=pl.BlockSpec((rows, d_out), lambda i: (i, 0)),
        out_shape=jax.ShapeDtypeStruct((vocab, d_out), jnp.float32),
    )(emb_t)


def _mm_body(h0_ref, h1_ref, w0_ref, w1_ref, b_ref, o_ref):
    acc = lax.dot_general(
        h0_ref[...], w0_ref[...],
        dimension_numbers=(((1,), (1,)), ((), ())),
        preferred_element_type=jnp.float32,
    )
    acc += lax.dot_general(
        h1_ref[...], w1_ref[...],
        dimension_numbers=(((1,), (1,)), ((), ())),
        preferred_element_type=jnp.float32,
    )
    o_ref[...] = acc + b_ref[...]


def kernel(x, emb, W, b):
    seq, batch = x.shape
    vocab, d = emb.shape
    n_class = W.shape[0]
    d_out = 160
    split = d_out

    idx = jnp.transpose(x).reshape(batch, 2, seq // 2)
    emb_t = jnp.transpose(emb)
    c0 = _pad_chunk(emb_t, 0, split, d_out)
    gm = _build_gather_max(vocab, d_out, batch, seq)
    h0 = gm(c0, idx)
    c1 = _pad_chunk(emb_t, split, d, d_out)
    h1 = gm(c1, idx)

    w0 = W[:, :split]
    w1 = jnp.pad(W[:, split:], ((0, 0), (0, d_out - (d - split))))

    return pl.pallas_call(
        _mm_body,
        out_shape=jax.ShapeDtypeStruct((batch, n_class), jnp.float32),
    )(h0, h1, w0, w1, b.reshape(1, n_class))


# speedup vs baseline: 1.1210x; 1.1210x over previous
import functools

import jax
import jax.numpy as jnp
from jax import lax
from jax.experimental import pallas as pl
from jax.experimental.pallas import tpu as pltpu
from jax.experimental.pallas import tpu_sc as plsc

NUM_CORES = 2
NUM_SUBCORES = 16
NUM_WORKERS = NUM_CORES * NUM_SUBCORES
LANES = 16


@functools.lru_cache(maxsize=None)
def _build_gather_max(vocab, d_pad, batch, seq):
    half = seq // 2
    b_per_w = batch // NUM_WORKERS
    n_chunks = d_pad // LANES
    mesh = plsc.VectorSubcoreMesh(core_axis_name="c", subcore_axis_name="s")

    @functools.partial(
        pl.kernel,
        mesh=mesh,
        compiler_params=pltpu.CompilerParams(use_tc_tiling_on_sc=False),
        out_type=jax.ShapeDtypeStruct((batch, d_pad), jnp.float32),
        scratch_types=[
            pltpu.VMEM((2, 2, half), jnp.int32),
            pltpu.VMEM((2, 2, half, d_pad), jnp.float32),
            pltpu.VMEM((d_pad,), jnp.float32),
            pltpu.SemaphoreType.DMA,
            pltpu.SemaphoreType.DMA,
        ],
    )
    def gather_max(emb_hbm, idx_hbm, out_hbm, idx_v, rows_v, hrow_v, sem0, sem1):
        wid = lax.axis_index("s") * NUM_CORES + lax.axis_index("c")
        base = wid * b_per_w
        sems = (sem0, sem1)

        def issue(buf, i):
            pltpu.sync_copy(idx_hbm.at[i], idx_v.at[buf])
            pltpu.async_copy(emb_hbm.at[idx_v.at[buf, 0]], rows_v.at[buf, 0],
                             sems[buf])
            pltpu.async_copy(emb_hbm.at[idx_v.at[buf, 1]], rows_v.at[buf, 1],
                             sems[buf])

        def drain(buf):
            pltpu.make_async_copy(emb_hbm.at[idx_v.at[buf, 0]],
                                  rows_v.at[buf, 0], sems[buf]).wait()
            pltpu.make_async_copy(emb_hbm.at[idx_v.at[buf, 1]],
                                  rows_v.at[buf, 1], sems[buf]).wait()

        def compute_store(buf, i):
            r0 = rows_v.at[buf, 0]
            r1 = rows_v.at[buf, 1]

            def body(r, carry):
                out = []
                for c in range(n_chunks):
                    sl = pl.ds(c * LANES, LANES)
                    m = jnp.maximum(r0[r, sl], r1[r, sl])
                    out.append(jnp.maximum(carry[c], m))
                return tuple(out)

            neg_inf = jnp.full((LANES,), -jnp.inf, jnp.float32)
            acc = lax.fori_loop(0, half, body, (neg_inf,) * n_chunks)
            for c in range(n_chunks):
                hrow_v[pl.ds(c * LANES, LANES)] = acc[c]
            pltpu.sync_copy(hrow_v, out_hbm.at[i])

        issue(0, base)

        def loop_body(g, carry):
            i0 = base + 2 * g
            issue(1, i0 + 1)
            drain(0)
            compute_store(0, i0)

            @pl.when(g < b_per_w // 2 - 1)
            def _():
                issue(0, i0 + 2)

            drain(1)
            compute_store(1, i0 + 1)
            return carry

        lax.fori_loop(0, b_per_w // 2, loop_body, 0)

    return gather_max


def _tp_body(e_ref, o_ref):
    t = jnp.transpose(e_ref[...], (1, 0))
    n, d = t.shape
    o_ref[...] = jnp.concatenate(
        [t, jnp.zeros((n, o_ref.shape[1] - d), jnp.float32)], axis=1)


def _pad_table(emb_t, d_pad):
    d, vocab = emb_t.shape
    rows = 8192
    return pl.pallas_call(
        _tp_body,
        grid=(-(-vocab // rows),),
        in_specs=[pl.BlockSpec((d, rows), lambda i: (0, i))],
        out_specs=pl.BlockSpec((rows, d_pad), lambda i: (i, 0)),
        out_shape=jax.ShapeDtypeStruct((vocab, d_pad), jnp.float32),
    )(emb_t)


def _mm_body(h_ref, w_ref, b_ref, o_ref):
    o_ref[...] = lax.dot_general(
        h_ref[...], w_ref[...],
        dimension_numbers=(((1,), (1,)), ((), ())),
        preferred_element_type=jnp.float32,
    ) + b_ref[...]


def kernel(x, emb, W, b):
    seq, batch = x.shape
    vocab, d = emb.shape
    n_class = W.shape[0]
    d_pad = -(-d // LANES) * LANES

    idx = jnp.transpose(x).reshape(batch, 2, seq // 2)
    emb_p = _pad_table(jnp.transpose(emb), d_pad)
    w_p = jnp.pad(W, ((0, 0), (0, d_pad - d)))

    h = _build_gather_max(vocab, d_pad, batch, seq)(emb_p, idx)

    return pl.pallas_call(
        _mm_body,
        out_shape=jax.ShapeDtypeStruct((batch, n_class), jnp.float32),
    )(h, w_p, b.reshape(1, n_class))


# speedup vs baseline: 1.3285x; 1.1850x over previous
import functools

import jax
import jax.numpy as jnp
from jax import lax
from jax.experimental import pallas as pl
from jax.experimental.pallas import tpu as pltpu
from jax.experimental.pallas import tpu_sc as plsc

NUM_CORES = 2
NUM_SUBCORES = 16
NUM_WORKERS = NUM_CORES * NUM_SUBCORES
LANES = 32


@functools.lru_cache(maxsize=None)
def _build_gather_max(vocab, d_pad, batch, seq):
    half = seq // 2
    b_per_w = batch // NUM_WORKERS
    n_chunks = d_pad // LANES
    mesh = plsc.VectorSubcoreMesh(core_axis_name="c", subcore_axis_name="s")

    @functools.partial(
        pl.kernel,
        mesh=mesh,
        compiler_params=pltpu.CompilerParams(use_tc_tiling_on_sc=False),
        out_type=jax.ShapeDtypeStruct((batch, d_pad), jnp.bfloat16),
        scratch_types=[
            pltpu.VMEM((2, 2, half), jnp.int32),
            pltpu.VMEM((2, 2, half, d_pad), jnp.bfloat16),
            pltpu.VMEM((d_pad,), jnp.bfloat16),
            pltpu.SemaphoreType.DMA,
            pltpu.SemaphoreType.DMA,
        ],
    )
    def gather_max(emb_hbm, idx_hbm, out_hbm, idx_v, rows_v, hrow_v, sem0, sem1):
        wid = lax.axis_index("s") * NUM_CORES + lax.axis_index("c")
        base = wid * b_per_w
        sems = (sem0, sem1)

        def issue(buf, i):
            pltpu.sync_copy(idx_hbm.at[i], idx_v.at[buf])
            pltpu.async_copy(emb_hbm.at[idx_v.at[buf, 0]], rows_v.at[buf, 0],
                             sems[buf])
            pltpu.async_copy(emb_hbm.at[idx_v.at[buf, 1]], rows_v.at[buf, 1],
                             sems[buf])

        def drain(buf):
            pltpu.make_async_copy(emb_hbm.at[idx_v.at[buf, 0]],
                                  rows_v.at[buf, 0], sems[buf]).wait()
            pltpu.make_async_copy(emb_hbm.at[idx_v.at[buf, 1]],
                                  rows_v.at[buf, 1], sems[buf]).wait()

        def compute_store(buf, i):
            r0 = rows_v.at[buf, 0]
            r1 = rows_v.at[buf, 1]

            def body(r, carry):
                out = []
                for c in range(n_chunks):
                    sl = pl.ds(c * LANES, LANES)
                    m = jnp.maximum(r0[r, sl], r1[r, sl])
                    out.append(jnp.maximum(carry[c], m))
                return tuple(out)

            neg_inf = jnp.full((LANES,), -jnp.inf, jnp.bfloat16)
            acc = lax.fori_loop(0, half, body, (neg_inf,) * n_chunks)
            for c in range(n_chunks):
                hrow_v[pl.ds(c * LANES, LANES)] = acc[c]
            pltpu.sync_copy(hrow_v, out_hbm.at[i])

        issue(0, base)

        def loop_body(g, carry):
            i0 = base + 2 * g
            issue(1, i0 + 1)
            drain(0)
            compute_store(0, i0)

            @pl.when(g < b_per_w // 2 - 1)
            def _():
                issue(0, i0 + 2)

            drain(1)
            compute_store(1, i0 + 1)
            return carry

        lax.fori_loop(0, b_per_w // 2, loop_body, 0)

    return gather_max


def _tp_body(e_ref, o_ref):
    t = jnp.transpose(e_ref[...], (1, 0))
    n, d = t.shape
    t = jnp.concatenate(
        [t, jnp.zeros((n, o_ref.shape[1] - d), jnp.float32)], axis=1)
    o_ref[...] = t.astype(jnp.bfloat16)


def _pad_table(emb_t, d_pad):
    d, vocab = emb_t.shape
    rows = 8192
    return pl.pallas_call(
        _tp_body,
        grid=(-(-vocab // rows),),
        in_specs=[pl.BlockSpec((d, rows), lambda i: (0, i))],
        out_specs=pl.BlockSpec((rows, d_pad), lambda i: (i, 0)),
        out_shape=jax.ShapeDtypeStruct((vocab, d_pad), jnp.bfloat16),
    )(emb_t)


def _mm_body(h_ref, w_ref, b_ref, o_ref):
    o_ref[...] = lax.dot_general(
        h_ref[...].astype(jnp.float32), w_ref[...],
        dimension_numbers=(((1,), (1,)), ((), ())),
        preferred_element_type=jnp.float32,
    ) + b_ref[...]


def kernel(x, emb, W, b):
    seq, batch = x.shape
    vocab, d = emb.shape
    n_class = W.shape[0]
    d_pad = -(-d // LANES) * LANES

    idx = jnp.transpose(x).reshape(batch, 2, seq // 2)
    emb_p = _pad_table(jnp.transpose(emb), d_pad)
    w_p = jnp.pad(W, ((0, 0), (0, d_pad - d)))

    h = _build_gather_max(vocab, d_pad, batch, seq)(emb_p, idx)

    return pl.pallas_call(
        _mm_body,
        out_shape=jax.ShapeDtypeStruct((batch, n_class), jnp.float32),
    )(h, w_p, b.reshape(1, n_class))
